# baseline (speedup 1.0000x reference)
import jax
import jax.numpy as jnp
from jax import lax
from jax.experimental import pallas as pl
from jax.experimental.pallas import tpu as pltpu
from jax.experimental.pallas import tpu_sc as plsc

_BPW = 4
_GS = 128
_NBUF = 2
_M = 2 * _NBUF
_NC = 2
_NS = 16
_NWORKERS = _NC * _NS


def _embedding_gather(flat_idx, table, batch, hist, d_model):
    num_indices = batch * hist
    per_worker = num_indices // _NWORKERS
    batches_per_worker = batch // _NWORKERS
    nwin = batches_per_worker // _BPW
    win_idx = _BPW * hist
    mesh = plsc.VectorSubcoreMesh(core_axis_name="core",
                                  subcore_axis_name="subcore")

    @pl.kernel(
        out_type=jax.ShapeDtypeStruct((batch, hist, d_model), table.dtype),
        mesh=mesh,
        scratch_types=[
            pltpu.VMEM((per_worker,), jnp.int32),
            pltpu.VMEM((_M, win_idx, d_model), table.dtype),
            pltpu.SemaphoreType.DMA((_M,)),
            pltpu.SemaphoreType.DMA((_M,)),
        ],
        compiler_params=pltpu.CompilerParams(use_tc_tiling_on_sc=False),
    )
    def gather_kernel(table_hbm, idx_hbm, out_hbm, idx_v, rows_v, gsem, ssem):
        wid = lax.axis_index("subcore") * _NC + lax.axis_index("core")
        idx_base = wid * per_worker
        b_base = wid * batches_per_worker
        pltpu.sync_copy(idx_hbm.at[pl.ds(idx_base, per_worker)], idx_v)

        def gather_part(w, slot, off, n):
            return pltpu.make_async_copy(
                table_hbm.at[idx_v.at[pl.ds(w * win_idx + off, n)]],
                rows_v.at[slot].at[pl.ds(off, n)],
                gsem.at[slot],
            )

        def gather_start(w, slot):
            gather_part(w, slot, 0, _GS).start()
            gather_part(w, slot, _GS, win_idx - _GS).start()

        def gather_wait(slot):
            gather_part(0, slot, 0, _GS).wait()
            gather_part(0, slot, _GS, win_idx - _GS).wait()

        def store_part(w, slot, part):
            return pltpu.make_async_copy(
                rows_v.at[slot].at[pl.ds(part * hist, hist)],
                out_hbm.at[b_base + w * _BPW + part],
                ssem.at[slot],
            )

        def store_start(w, slot):
            for part in range(_BPW):
                store_part(w, slot, part).start()

        def store_wait(slot):
            for part in range(_BPW):
                store_part(0, slot, part).wait()

        for w in range(_NBUF):
            gather_start(w, w)

        @pl.loop(0, nwin, step=_M)
        def _(g):
            for j in range(_M):
                w = g + j
                gather_wait(j)
                store_start(w, j)
                v = w + _NBUF
                slot = (j + _NBUF) % _M

                @pl.when(jnp.logical_and(v < nwin, v >= _M))
                def _():
                    store_wait(slot)

                @pl.when(v < nwin)
                def _():
                    gather_start(v, slot)

        for j in range(_M):
            store_wait(j)

    return gather_kernel(table, flat_idx)


def kernel(x, table):
    batch, hist = x.shape
    vocab, d_model = table.shape
    flat_idx = x.reshape(batch * hist).astype(jnp.int32)
    return _embedding_gather(flat_idx, table, batch, hist, d_model)


# speedup vs baseline: 1.3485x; 1.3485x over previous
import jax
import jax.numpy as jnp
from jax import lax
from jax.experimental import pallas as pl
from jax.experimental.pallas import tpu as pltpu
from jax.experimental.pallas import tpu_sc as plsc

_BPW = 4
_GS = 128
_NBUF = 2
_M = 2 * _NBUF
_NC = 2
_NS = 16
_NWORKERS = _NC * _NS
_HPAD = 56
_DPAD = 128


def _embedding_gather(flat_idx, table, batch, hist, d_model):
    num_indices = batch * hist
    per_worker = num_indices // _NWORKERS
    batches_per_worker = batch // _NWORKERS
    nwin = batches_per_worker // _BPW
    win_idx = _BPW * hist
    mesh = plsc.VectorSubcoreMesh(core_axis_name="core",
                                  subcore_axis_name="subcore")

    @pl.kernel(
        out_type=jax.ShapeDtypeStruct((batch, _HPAD, _DPAD), table.dtype),
        mesh=mesh,
        scratch_types=[
            pltpu.VMEM((per_worker,), jnp.int32),
            pltpu.VMEM((_M, win_idx, d_model), table.dtype),
            pltpu.SemaphoreType.DMA((_M,)),
            pltpu.SemaphoreType.DMA((_M,)),
        ],
        compiler_params=pltpu.CompilerParams(use_tc_tiling_on_sc=False),
    )
    def gather_kernel(table_hbm, idx_hbm, out_hbm, idx_v, rows_v, gsem, ssem):
        wid = lax.axis_index("subcore") * _NC + lax.axis_index("core")
        idx_base = wid * per_worker
        b_base = wid * batches_per_worker
        pltpu.sync_copy(idx_hbm.at[pl.ds(idx_base, per_worker)], idx_v)

        def gather_part(w, slot, off, n):
            return pltpu.make_async_copy(
                table_hbm.at[idx_v.at[pl.ds(w * win_idx + off, n)]],
                rows_v.at[slot].at[pl.ds(off, n)],
                gsem.at[slot],
            )

        def gather_start(w, slot):
            gather_part(w, slot, 0, _GS).start()
            gather_part(w, slot, _GS, win_idx - _GS).start()

        def gather_wait(slot):
            gather_part(0, slot, 0, _GS).wait()
            gather_part(0, slot, _GS, win_idx - _GS).wait()

        def store_part(w, slot, part):
            return pltpu.make_async_copy(
                rows_v.at[slot].at[pl.ds(part * hist, hist)],
                out_hbm.at[b_base + w * _BPW + part,
                           pl.ds(0, hist), pl.ds(0, d_model)],
                ssem.at[slot],
            )

        def store_start(w, slot):
            for part in range(_BPW):
                store_part(w, slot, part).start()

        def store_wait(slot):
            for part in range(_BPW):
                store_part(0, slot, part).wait()

        for w in range(_NBUF):
            gather_start(w, w)

        @pl.loop(0, nwin, step=_M)
        def _(g):
            for j in range(_M):
                w = g + j
                gather_wait(j)
                store_start(w, j)
                v = w + _NBUF
                slot = (j + _NBUF) % _M

                @pl.when(jnp.logical_and(v < nwin, v >= _M))
                def _():
                    store_wait(slot)

                @pl.when(v < nwin)
                def _():
                    gather_start(v, slot)

        for j in range(_M):
            store_wait(j)

    return gather_kernel(table, flat_idx)


def kernel(x, table):
    batch, hist = x.shape
    vocab, d_model = table.shape
    flat_idx = x.reshape(batch * hist).astype(jnp.int32)
    out_pad = _embedding_gather(flat_idx, table, batch, hist, d_model)
    return out_pad[:, :hist, :d_model]
